# Initial kernel scaffold; baseline (speedup 1.0000x reference)
import jax
import jax.numpy as jnp
from jax.experimental import pallas as pl


def kernel(input_ids, embed_tokens):
    raise NotImplementedError("write your pallas kernel here")


# speedup vs baseline: 4.9373x; 4.9373x over previous
import functools

import jax
import jax.numpy as jnp
from jax import lax
from jax.experimental import pallas as pl
from jax.experimental.pallas import tpu as pltpu
from jax.experimental.pallas import tpu_sc as plsc

VOCAB = 32
D = 8
B, T = 4096, 200
N = B * T
NC, NS, L = 2, 16, 16
NW = NC * NS
PER_W = N // NW
CHUNK = 3200
GROUPS = CHUNK // L
NCHUNK = PER_W // CHUNK

_mesh = plsc.VectorSubcoreMesh(core_axis_name="c", subcore_axis_name="s")


@functools.partial(
    pl.kernel,
    mesh=_mesh,
    out_type=jax.ShapeDtypeStruct((N * D,), jnp.float32),
    scratch_types=[
        pltpu.VMEM((VOCAB * D,), jnp.float32),
        pltpu.VMEM((CHUNK,), jnp.int32),
        pltpu.VMEM((CHUNK * D,), jnp.float32),
    ],
    compiler_params=pltpu.CompilerParams(needs_layout_passes=False),
)
def _embed_lookup(ids_hbm, table_hbm, out_hbm, table_v, idx_v, rows_v):
    wid = lax.axis_index("s") * NC + lax.axis_index("c")
    base = wid * PER_W

    pltpu.sync_copy(table_hbm, table_v)

    lane8 = lax.iota(jnp.int32, L) * D
    pos_j = [lane8 + j for j in range(D)]

    def chunk_body(c, carry):
        off = pl.multiple_of(base + c * CHUNK, CHUNK)
        pltpu.sync_copy(ids_hbm.at[pl.ds(off, CHUNK)], idx_v)

        def group_body(g, carry2):
            idsv = idx_v[pl.ds(pl.multiple_of(g * L, L), L)]
            gbase = idsv * D
            obase = g * (L * D)
            for j in range(D):
                vals = plsc.load_gather(table_v, [gbase + j])
                plsc.store_scatter(rows_v, [pos_j[j] + obase], vals)
            return carry2

        lax.fori_loop(0, GROUPS, group_body, 0)
        pltpu.sync_copy(rows_v, out_hbm.at[pl.ds(off * D, CHUNK * D)])
        return carry

    lax.fori_loop(0, NCHUNK, chunk_body, 0)


def kernel(input_ids, embed_tokens):
    ids = input_ids.reshape(-1).astype(jnp.int32)
    out = _embed_lookup(ids, embed_tokens.reshape(-1))
    return out.reshape(B, T, D)


# speedup vs baseline: 20.8188x; 4.2167x over previous
import functools

import jax
import jax.numpy as jnp
from jax import lax
from jax.experimental import pallas as pl
from jax.experimental.pallas import tpu as pltpu
from jax.experimental.pallas import tpu_sc as plsc

VOCAB = 32
D = 8
B, T = 4096, 200
NC, NS, L = 2, 16, 16
NW = NC * NS
GROUPS = B // L
BIG = T - (T // NW) * NW
SLABS_BIG = T // NW + 1
SLABS_SMALL = T // NW

_mesh = plsc.VectorSubcoreMesh(core_axis_name="c", subcore_axis_name="s")


@functools.partial(
    pl.kernel,
    mesh=_mesh,
    out_type=jax.ShapeDtypeStruct((T, D, B), jnp.float32),
    scratch_types=[
        pltpu.VMEM((VOCAB * D,), jnp.float32),
        pltpu.VMEM((B,), jnp.int32),
        pltpu.VMEM((D, B), jnp.float32),
    ],
    compiler_params=pltpu.CompilerParams(needs_layout_passes=False),
)
def _embed_lookup(ids_hbm, table_hbm, out_hbm, table_v, idx_v, slab_v):
    wid = lax.axis_index("s") * NC + lax.axis_index("c")
    t0 = jnp.where(wid < BIG, wid * SLABS_BIG,
                   BIG * SLABS_BIG + (wid - BIG) * SLABS_SMALL)
    nt = jnp.where(wid < BIG, SLABS_BIG, SLABS_SMALL)

    pltpu.sync_copy(table_hbm, table_v)

    def slab_body(k, carry):
        t = t0 + k
        pltpu.sync_copy(ids_hbm.at[pl.ds(t * B, B)], idx_v)

        def group_body(g, carry2):
            o = pl.multiple_of(g * L, L)
            gb = idx_v[pl.ds(o, L)] * D
            for j in range(D):
                slab_v[j, pl.ds(o, L)] = plsc.load_gather(table_v, [gb + j])
            return carry2

        lax.fori_loop(0, GROUPS, group_body, 0)
        pltpu.sync_copy(slab_v, out_hbm.at[t])
        return carry

    lax.fori_loop(0, nt, slab_body, 0)


def kernel(input_ids, embed_tokens):
    ids_t = input_ids.T.reshape(-1).astype(jnp.int32)
    out = _embed_lookup(ids_t, embed_tokens.reshape(-1))
    return out.transpose(2, 0, 1)


# speedup vs baseline: 40.3967x; 1.9404x over previous
import functools

import jax
import jax.numpy as jnp
from jax import lax
from jax.experimental import pallas as pl
from jax.experimental.pallas import tpu as pltpu
from jax.experimental.pallas import tpu_sc as plsc

VOCAB = 32
D = 8
B, T = 4096, 200
NC, NS, L = 2, 16, 16
NW = NC * NS
GROUPS = B // L
BIG = T - (T // NW) * NW
SLABS_BIG = T // NW + 1
SLABS_SMALL = T // NW

_mesh = plsc.VectorSubcoreMesh(core_axis_name="c", subcore_axis_name="s")


@functools.partial(
    pl.kernel,
    mesh=_mesh,
    out_type=jax.ShapeDtypeStruct((T, D, B), jnp.float32),
    scratch_types=[
        pltpu.VMEM((VOCAB * D,), jnp.float32),
        pltpu.VMEM((B,), jnp.int32),
        pltpu.VMEM((D, B), jnp.float32),
    ],
    compiler_params=pltpu.CompilerParams(needs_layout_passes=False),
)
def _embed_lookup(ids_hbm, table_hbm, out_hbm, table_v, idx_v, slab_v):
    wid = lax.axis_index("s") * NC + lax.axis_index("c")
    t0 = jnp.where(wid < BIG, wid * SLABS_BIG,
                   BIG * SLABS_BIG + (wid - BIG) * SLABS_SMALL)
    nt = jnp.where(wid < BIG, SLABS_BIG, SLABS_SMALL)

    pltpu.sync_copy(table_hbm, table_v)

    def slab_body(k, carry):
        t = t0 + k
        pltpu.sync_copy(ids_hbm.at[pl.ds(t * B, B)], idx_v)

        @plsc.parallel_loop(0, B, step=L, unroll=8)
        def group_body(i):
            o = pl.multiple_of(i, L)
            gb = idx_v[pl.ds(o, L)] * D
            for j in range(D):
                slab_v[j, pl.ds(o, L)] = plsc.load_gather(table_v, [gb + j])
        pltpu.sync_copy(slab_v, out_hbm.at[t])
        return carry

    lax.fori_loop(0, nt, slab_body, 0)


def kernel(input_ids, embed_tokens):
    ids_t = input_ids.T.reshape(-1).astype(jnp.int32)
    out = _embed_lookup(ids_t, embed_tokens.reshape(-1))
    return out.transpose(2, 0, 1)
